# baseline (speedup 1.0000x reference)
import functools

import jax
import jax.numpy as jnp
from jax import lax
from jax.experimental import pallas as pl
from jax.experimental.pallas import tpu as pltpu
from jax.experimental.pallas import tpu_sc as plsc

B, T, H, W = 4, 64, 224, 224
C = 4
CIN = 3
NSEL = 4
TB = 8


def _score_sums_kernel(wc_ref, bc_ref, x_ref, y_ref):
    t = pl.program_id(1)
    xb = x_ref[0]
    x0, x1, x2 = xb[0], xb[1], xb[2]
    chans = []
    for o in range(C):
        e = x2 * wc_ref[o, 2] + bc_ref[o]
        e = x1 * wc_ref[o, 1] + e
        e = x0 * wc_ref[o, 0] + e
        chans.append(jnp.maximum(e, 0.0))
    tot = (chans[0] + chans[1]) + (chans[2] + chans[3])
    lane = lax.broadcasted_iota(jnp.int32, (1, T), 1)
    row = jnp.zeros((1, T), dtype=jnp.float32)
    for i in range(TB):
        si = jnp.sum(tot[i])
        row = row + jnp.where(lane == t * TB + i, si, 0.0)

    @pl.when(t == 0)
    def _():
        y_ref[...] = jnp.zeros_like(y_ref)

    y_ref[...] = y_ref[...] + row[None]


def _select_kernel(y_ref, w1_ref, b1_ref, w2_ref, b2_ref, idx_ref):
    y = y_ref[...] * (1.0 / (C * H * W))
    h = jnp.maximum(
        lax.dot_general(y, w1_ref[...], (((1,), (1,)), ((), ())),
                        preferred_element_type=jnp.float32) + b1_ref[...], 0.0)
    s = jax.nn.sigmoid(
        lax.dot_general(h, w2_ref[...], (((1,), (1,)), ((), ())),
                        preferred_element_type=jnp.float32) + b2_ref[...])
    iota = lax.broadcasted_iota(jnp.int32, (B, T), 1)
    picks = []
    cur = s
    for _ in range(NSEL):
        m = jnp.max(cur, axis=1, keepdims=True)
        idxv = jnp.min(jnp.where(cur == m, iota, T), axis=1)
        picks.append(idxv)
        cur = jnp.where(iota == idxv[:, None], -1.0, cur)
    cur = s
    for _ in range(NSEL):
        m = jnp.min(cur, axis=1, keepdims=True)
        idxv = jnp.min(jnp.where(cur == m, iota, T), axis=1)
        picks.append(idxv)
        cur = jnp.where(iota == idxv[:, None], 2.0, cur)
    idx_ref[...] = jnp.stack(picks, axis=1)


def _gather_embed_kernel(idx_ref, wc_ref, bc_ref, x_ref, out_ref):
    del idx_ref
    xb = x_ref[0, :, 0]
    x0, x1, x2 = xb[0], xb[1], xb[2]
    for o in range(C):
        e = x2 * wc_ref[o, 2] + bc_ref[o]
        e = x1 * wc_ref[o, 1] + e
        e = x0 * wc_ref[o, 0] + e
        out_ref[0, o, 0] = jnp.maximum(e, 0.0)


NW = 32
CH = 16
NCH = H // CH
L = 16


def _sc_gather_embed(xp_hbm, idx_hbm, wb_hbm, out_hbm, idx_v, wb_v, xb, ob,
                     sin0, sin1, so0, so1):
    wid = lax.axis_index("s") * 2 + lax.axis_index("c")
    b = wid // (2 * NSEL)
    j = wid % (2 * NSEL)
    pltpu.sync_copy(idx_hbm.at[pl.ds(wid * L, L)], idx_v)
    pltpu.sync_copy(wb_hbm, wb_v)
    t = idx_v[...][0]
    w_vecs = [[wb_v[pl.ds((o * CIN + c) * L, L)] for c in range(CIN)]
              for o in range(C)]
    b_vecs = [wb_v[pl.ds((C * CIN + o) * L, L)] for o in range(C)]
    sin = [sin0, sin1]
    sout = [so0, so1]
    in_handles = [None] * NCH
    out_handles = [None] * NCH

    def fire_in(k):
        p = k % 2
        in_handles[k] = [
            pltpu.async_copy(
                xp_hbm.at[(b * CIN + c) * T + t, pl.ds(k * CH, CH)],
                xb.at[p, c], sin[p])
            for c in range(CIN)]

    fire_in(0)
    for k in range(NCH):
        p = k % 2
        for hnd in in_handles[k]:
            hnd.wait()
        if k >= 1:
            for hnd in out_handles[k - 1]:
                hnd.wait()
        if k + 1 < NCH:
            fire_in(k + 1)

        def row_body(r, carry, p=p):
            for l in range(W // L):
                sl = pl.ds(l * L, L)
                x0 = xb[p, 0, r, sl]
                x1 = xb[p, 1, r, sl]
                x2 = xb[p, 2, r, sl]
                for o in range(C):
                    e = x2 * w_vecs[o][2] + b_vecs[o]
                    e = x1 * w_vecs[o][1] + e
                    e = x0 * w_vecs[o][0] + e
                    ob[p, o, r, sl] = jnp.maximum(e, 0.0)
            return carry

        lax.fori_loop(0, CH, row_body, 0)
        out_handles[k] = [
            pltpu.async_copy(
                ob.at[p, o],
                out_hbm.at[(b * C + o) * 2 * NSEL + j, pl.ds(k * CH, CH)],
                sout[p])
            for o in range(C)]
    for hnd in out_handles[NCH - 1]:
        hnd.wait()


def _sc_gather_call(xp, idx1d, wb1d):
    mesh = plsc.VectorSubcoreMesh(core_axis_name="c", subcore_axis_name="s")
    f = functools.partial(
        pl.kernel,
        out_type=jax.ShapeDtypeStruct((B * C * 2 * NSEL, H, W), jnp.float32),
        mesh=mesh,
        scratch_types=[
            pltpu.VMEM((L,), jnp.int32),
            pltpu.VMEM((L * L,), jnp.float32),
            pltpu.VMEM((2, CIN, CH, W), jnp.float32),
            pltpu.VMEM((2, C, CH, W), jnp.float32),
            pltpu.SemaphoreType.DMA,
            pltpu.SemaphoreType.DMA,
            pltpu.SemaphoreType.DMA,
            pltpu.SemaphoreType.DMA,
        ],
    )(_sc_gather_embed)
    return f(xp, idx1d, wb1d)


def kernel(x, Wc, bc, W1, b1, W2, b2):
    y_sums = pl.pallas_call(
        _score_sums_kernel,
        grid=(B, T // TB),
        in_specs=[
            pl.BlockSpec(memory_space=pltpu.SMEM),
            pl.BlockSpec(memory_space=pltpu.SMEM),
            pl.BlockSpec((1, CIN, TB, H, W), lambda b, t: (b, 0, t, 0, 0)),
        ],
        out_specs=pl.BlockSpec((1, 1, T), lambda b, t: (b, 0, 0)),
        out_shape=jax.ShapeDtypeStruct((B, 1, T), jnp.float32),
    )(Wc, bc, x)

    idx = pl.pallas_call(
        _select_kernel,
        out_shape=jax.ShapeDtypeStruct((B, 2 * NSEL), jnp.int32),
    )(y_sums.reshape(B, T), W1, b1.reshape(1, 32), W2, b2.reshape(1, 64))

    xp = x.reshape(B * CIN * T, H, W)
    idx1d = jnp.broadcast_to(idx.reshape(NW, 1), (NW, L)).reshape(NW * L)
    wb1d = jnp.broadcast_to(
        jnp.concatenate([Wc.reshape(C * CIN), bc])[:, None], (L, L)).reshape(L * L)
    out = _sc_gather_call(xp, idx1d, wb1d)
    return out.reshape(B, C, 2 * NSEL, H, W)


# speedup vs baseline: 1.0551x; 1.0551x over previous
import functools

import jax
import jax.numpy as jnp
from jax import lax
from jax.experimental import pallas as pl
from jax.experimental.pallas import tpu as pltpu
from jax.experimental.pallas import tpu_sc as plsc

B, T, H, W = 4, 64, 224, 224
C = 4
CIN = 3
NSEL = 4
TB = 8


def _score_sums_kernel(wc_ref, bc_ref, x_ref, y_ref):
    t = pl.program_id(1)
    xb = x_ref[0]
    x0, x1, x2 = xb[0], xb[1], xb[2]
    chans = []
    for o in range(C):
        e = x2 * wc_ref[o, 2] + bc_ref[o]
        e = x1 * wc_ref[o, 1] + e
        e = x0 * wc_ref[o, 0] + e
        chans.append(jnp.maximum(e, 0.0))
    tot = (chans[0] + chans[1]) + (chans[2] + chans[3])
    lane = lax.broadcasted_iota(jnp.int32, (1, T), 1)
    row = jnp.zeros((1, T), dtype=jnp.float32)
    for i in range(TB):
        si = jnp.sum(tot[i])
        row = row + jnp.where(lane == t * TB + i, si, 0.0)

    @pl.when(t == 0)
    def _():
        y_ref[...] = jnp.zeros_like(y_ref)

    y_ref[...] = y_ref[...] + row[None]


def _select_kernel(y_ref, w1_ref, b1_ref, w2_ref, b2_ref, idx_ref):
    y = y_ref[...] * (1.0 / (C * H * W))
    h = jnp.maximum(
        lax.dot_general(y, w1_ref[...], (((1,), (1,)), ((), ())),
                        preferred_element_type=jnp.float32) + b1_ref[...], 0.0)
    s = jax.nn.sigmoid(
        lax.dot_general(h, w2_ref[...], (((1,), (1,)), ((), ())),
                        preferred_element_type=jnp.float32) + b2_ref[...])
    iota = lax.broadcasted_iota(jnp.int32, (B, T), 1)
    picks = []
    cur = s
    for _ in range(NSEL):
        m = jnp.max(cur, axis=1, keepdims=True)
        idxv = jnp.min(jnp.where(cur == m, iota, T), axis=1)
        picks.append(idxv)
        cur = jnp.where(iota == idxv[:, None], -1.0, cur)
    cur = s
    for _ in range(NSEL):
        m = jnp.min(cur, axis=1, keepdims=True)
        idxv = jnp.min(jnp.where(cur == m, iota, T), axis=1)
        picks.append(idxv)
        cur = jnp.where(iota == idxv[:, None], 2.0, cur)
    idx_ref[...] = jnp.stack(picks, axis=1)


def _gather_embed_kernel(idx_ref, wc_ref, bc_ref, x_ref, out_ref):
    del idx_ref
    xb = x_ref[0, :, 0]
    x0, x1, x2 = xb[0], xb[1], xb[2]
    for o in range(C):
        e = x2 * wc_ref[o, 2] + bc_ref[o]
        e = x1 * wc_ref[o, 1] + e
        e = x0 * wc_ref[o, 0] + e
        out_ref[0, o, 0] = jnp.maximum(e, 0.0)


NW = 32
CH = 16
NCH = H // CH
L = 16


def _sc_gather_embed(xp_hbm, idx_hbm, wb_hbm, out_hbm, idx_v, wb_v, xb, ob,
                     sin0, sin1, so0, so1, so2):
    wid = lax.axis_index("s") * 2 + lax.axis_index("c")
    b = wid // (2 * NSEL)
    j = wid % (2 * NSEL)
    pltpu.sync_copy(idx_hbm.at[pl.ds(wid * L, L)], idx_v)
    pltpu.sync_copy(wb_hbm, wb_v)
    t = idx_v[...][0]
    w_vecs = [[wb_v[pl.ds((o * CIN + c) * L, L)] for c in range(CIN)]
              for o in range(C)]
    b_vecs = [wb_v[pl.ds((C * CIN + o) * L, L)] for o in range(C)]
    sin = [sin0, sin1]
    sout = [so0, so1, so2]
    in_handles = [None] * NCH
    out_handles = [None] * NCH

    def fire_in(k):
        p = k % 2
        in_handles[k] = [
            pltpu.async_copy(
                xp_hbm.at[(b * CIN + c) * T + t, pl.ds(k * CH, CH)],
                xb.at[p, c], sin[p])
            for c in range(CIN)]

    fire_in(0)
    for k in range(NCH):
        p = k % 2
        q = k % 3
        for hnd in in_handles[k]:
            hnd.wait()
        if k + 1 < NCH:
            fire_in(k + 1)
        if k >= 3:
            for hnd in out_handles[k - 3]:
                hnd.wait()

        def row_body(r, carry, p=p, q=q):
            for l in range(W // L):
                sl = pl.ds(l * L, L)
                x0 = xb[p, 0, r, sl]
                x1 = xb[p, 1, r, sl]
                x2 = xb[p, 2, r, sl]
                for o in range(C):
                    e = x2 * w_vecs[o][2] + b_vecs[o]
                    e = x1 * w_vecs[o][1] + e
                    e = x0 * w_vecs[o][0] + e
                    ob[q, o, r, sl] = jnp.maximum(e, 0.0)
            return carry

        lax.fori_loop(0, CH, row_body, 0, unroll=2)
        out_handles[k] = [
            pltpu.async_copy(
                ob.at[q, o],
                out_hbm.at[(b * C + o) * 2 * NSEL + j, pl.ds(k * CH, CH)],
                sout[q])
            for o in range(C)]
    for k in (NCH - 3, NCH - 2, NCH - 1):
        for hnd in out_handles[k]:
            hnd.wait()


def _sc_gather_call(xp, idx1d, wb1d):
    mesh = plsc.VectorSubcoreMesh(core_axis_name="c", subcore_axis_name="s")
    f = functools.partial(
        pl.kernel,
        out_type=jax.ShapeDtypeStruct((B * C * 2 * NSEL, H, W), jnp.float32),
        mesh=mesh,
        scratch_types=[
            pltpu.VMEM((L,), jnp.int32),
            pltpu.VMEM((L * L,), jnp.float32),
            pltpu.VMEM((2, CIN, CH, W), jnp.float32),
            pltpu.VMEM((3, C, CH, W), jnp.float32),
            pltpu.SemaphoreType.DMA,
            pltpu.SemaphoreType.DMA,
            pltpu.SemaphoreType.DMA,
            pltpu.SemaphoreType.DMA,
            pltpu.SemaphoreType.DMA,
        ],
    )(_sc_gather_embed)
    return f(xp, idx1d, wb1d)


def kernel(x, Wc, bc, W1, b1, W2, b2):
    y_sums = pl.pallas_call(
        _score_sums_kernel,
        grid=(B, T // TB),
        in_specs=[
            pl.BlockSpec(memory_space=pltpu.SMEM),
            pl.BlockSpec(memory_space=pltpu.SMEM),
            pl.BlockSpec((1, CIN, TB, H, W), lambda b, t: (b, 0, t, 0, 0)),
        ],
        out_specs=pl.BlockSpec((1, 1, T), lambda b, t: (b, 0, 0)),
        out_shape=jax.ShapeDtypeStruct((B, 1, T), jnp.float32),
    )(Wc, bc, x)

    idx = pl.pallas_call(
        _select_kernel,
        out_shape=jax.ShapeDtypeStruct((B, 2 * NSEL), jnp.int32),
    )(y_sums.reshape(B, T), W1, b1.reshape(1, 32), W2, b2.reshape(1, 64))

    xp = x.reshape(B * CIN * T, H, W)
    idx1d = jnp.broadcast_to(idx.reshape(NW, 1), (NW, L)).reshape(NW * L)
    wb1d = jnp.broadcast_to(
        jnp.concatenate([Wc.reshape(C * CIN), bc])[:, None], (L, L)).reshape(L * L)
    out = _sc_gather_call(xp, idx1d, wb1d)
    return out.reshape(B, C, 2 * NSEL, H, W)
